# Initial kernel scaffold; baseline (speedup 1.0000x reference)
import jax
import jax.numpy as jnp
from jax.experimental import pallas as pl


def kernel(inputs, embeddings):
    raise NotImplementedError("write your pallas kernel here")


# speedup vs baseline: 2.9027x; 2.9027x over previous
import functools

import jax
import jax.numpy as jnp
from jax import lax
from jax.experimental import pallas as pl
from jax.experimental.pallas import tpu as pltpu
from jax.experimental.pallas import tpu_sc as plsc

BATCH_N = 16384
HIST_N = 50
DIM_N = 32

NUM_CORES = 2
NUM_SUBCORES = 16
NUM_WORKERS = NUM_CORES * NUM_SUBCORES


def _build(batch=BATCH_N, hist=HIST_N, dim=DIM_N, rows_per_chunk=32,
           group=100, interpret=False):
    assert batch % NUM_WORKERS == 0
    b_per_w = batch // NUM_WORKERS
    assert b_per_w % rows_per_chunk == 0
    chunks = b_per_w // rows_per_chunk
    assert (rows_per_chunk * hist) % group == 0 and group <= 128
    groups_per_chunk = rows_per_chunk * hist // group
    rows_buf = groups_per_chunk * group

    mesh = plsc.VectorSubcoreMesh(
        core_axis_name="c", subcore_axis_name="s",
        num_cores=NUM_CORES, num_subcores=NUM_SUBCORES)

    @functools.partial(
        pl.kernel,
        out_type=jax.ShapeDtypeStruct((batch, dim), jnp.float32),
        mesh=mesh,
        scratch_types=[
            pltpu.VMEM((2, groups_per_chunk, group), jnp.int32),
            pltpu.VMEM((2, rows_buf, dim), jnp.float32),
            pltpu.VMEM((2, rows_per_chunk, dim), jnp.float32),
            pltpu.SemaphoreType.DMA,
            pltpu.SemaphoreType.DMA,
        ],
        compiler_params=pltpu.CompilerParams(use_tc_tiling_on_sc=False),
        interpret=interpret,
    )
    def _sc_kernel(idx_hbm, table_hbm, out_hbm, idx_v, rows_v, out_v,
                   sem0, sem1):
        sems = (sem0, sem1)
        wid = lax.axis_index("s") * NUM_CORES + lax.axis_index("c")
        gbase = wid * (chunks * groups_per_chunk)
        bbase = wid * b_per_w

        def fire(c, b):
            pltpu.sync_copy(
                idx_hbm.at[pl.ds(gbase + c * groups_per_chunk,
                                 groups_per_chunk), :],
                idx_v.at[b])
            return [
                pltpu.async_copy(
                    table_hbm.at[idx_v.at[b, j]],
                    rows_v.at[b, pl.ds(j * group, group)],
                    sems[b])
                for j in range(groups_per_chunk)
            ]

        def accumulate(c, b):
            def body(r, carry):
                base = r * hist
                a0 = rows_v[b, base, pl.ds(0, 16)]
                a1 = rows_v[b, base, pl.ds(16, 16)]
                for l in range(1, hist):
                    a0 = a0 + rows_v[b, base + l, pl.ds(0, 16)]
                    a1 = a1 + rows_v[b, base + l, pl.ds(16, 16)]
                out_v[b, r, pl.ds(0, 16)] = a0
                out_v[b, r, pl.ds(16, 16)] = a1
                return carry
            lax.fori_loop(0, rows_per_chunk, body, 0)
            pltpu.sync_copy(
                out_v.at[b],
                out_hbm.at[pl.ds(bbase + c * rows_per_chunk,
                                 rows_per_chunk), :])

        handles = fire(0, 0)
        for c in range(chunks):
            next_handles = fire(c + 1, (c + 1) % 2) if c + 1 < chunks else ()
            for h in handles:
                h.wait()
            accumulate(c, c % 2)
            handles = next_handles

    def run(inputs, embeddings):
        idx = inputs.astype(jnp.int32).reshape(batch * hist // group, group)
        return _sc_kernel(idx, embeddings)

    return run


_kernel_impl = _build()


def kernel(inputs, embeddings):
    return _kernel_impl(inputs, embeddings)
